# Initial kernel scaffold; baseline (speedup 1.0000x reference)
import jax
import jax.numpy as jnp
from jax.experimental import pallas as pl


def kernel(node, edges, edges_attr, W1, b1, conv_W, conv_b, ln_g, ln_b, fc_W, fc_b):
    raise NotImplementedError("write your pallas kernel here")


# speedup vs baseline: 5.3183x; 5.3183x over previous
import functools

import jax
import jax.numpy as jnp
from jax import lax
from jax.experimental import pallas as pl
from jax.experimental.pallas import tpu as pltpu
from jax.experimental.pallas import tpu_sc as plsc

N = 10000
E = 320000
H = 128
STEPS = 3

NC = 2
NS = 16
NW = NC * NS
L = 16

K = 80
C = 129
EPT = C * K
EP = NW * EPT
C2 = 258
EPT2 = C2 * K
HH = H // 2

DEG_PAD = 10240
ROWS_PER_TILE = DEG_PAD // NS

_MESH = plsc.VectorSubcoreMesh(
    core_axis_name="c", subcore_axis_name="s", num_cores=NC, num_subcores=NS)


def _wid():
    return lax.axis_index("c") * NS + lax.axis_index("s")


def _deg_body(col_hbm, ew_hbm, out_hbm, col_v, ew_v, deg_v):
    w = _wid()
    pltpu.sync_copy(col_hbm.at[w], col_v)
    pltpu.sync_copy(ew_hbm.at[w], ew_v)

    @pl.loop(0, DEG_PAD // L)
    def _zero(i):
        deg_v[pl.ds(i * L, L)] = jnp.zeros((L,), jnp.float32)

    @pl.loop(0, EPT // L)
    def _acc(i):
        sl = pl.ds(i * L, L)
        plsc.addupdate_scatter(deg_v, [col_v[sl]], ew_v[sl])

    pltpu.sync_copy(deg_v, out_hbm.at[w])


_deg_kernel = pl.kernel(
    _deg_body,
    out_type=jax.ShapeDtypeStruct((NW, DEG_PAD), jnp.float32),
    mesh=_MESH,
    compiler_params=pltpu.CompilerParams(needs_layout_passes=False),
    scratch_types=[
        pltpu.VMEM((EPT,), jnp.int32),
        pltpu.VMEM((EPT,), jnp.float32),
        pltpu.VMEM((DEG_PAD,), jnp.float32),
    ],
)


def _dis_body(degp_ref, dis_ref):
    deg = jnp.sum(degp_ref[...], axis=0)
    dis_ref[...] = jnp.where(deg > 0.0, lax.rsqrt(deg), 0.0)


def _dis_kernel(degp):
    return pl.pallas_call(
        _dis_body,
        out_shape=jax.ShapeDtypeStruct((DEG_PAD // H, H), jnp.float32),
    )(degp.reshape(NW, DEG_PAD // H, H))


def _norm_body(dis_hbm, row_hbm, col_hbm, ew_hbm, out_hbm,
               dis_v, row_v, col_v, ew_v, norm_v):
    w = _wid()
    pltpu.sync_copy(dis_hbm, dis_v)
    pltpu.sync_copy(row_hbm.at[w], row_v)
    pltpu.sync_copy(col_hbm.at[w], col_v)
    pltpu.sync_copy(ew_hbm.at[w], ew_v)

    @pl.loop(0, EPT // L)
    def _go(i):
        sl = pl.ds(i * L, L)
        gr = plsc.load_gather(dis_v, [row_v[sl]])
        gc = plsc.load_gather(dis_v, [col_v[sl]])
        norm_v[sl] = gr * ew_v[sl] * gc

    pltpu.sync_copy(norm_v, out_hbm.at[w])


_norm_kernel = pl.kernel(
    _norm_body,
    out_type=jax.ShapeDtypeStruct((NW, EPT), jnp.float32),
    mesh=_MESH,
    compiler_params=pltpu.CompilerParams(needs_layout_passes=False),
    scratch_types=[
        pltpu.VMEM((DEG_PAD,), jnp.float32),
        pltpu.VMEM((EPT,), jnp.int32),
        pltpu.VMEM((EPT,), jnp.int32),
        pltpu.VMEM((EPT,), jnp.float32),
        pltpu.VMEM((EPT,), jnp.float32),
    ],
)


def _scat_body(xw0_hbm, xw1_hbm, row_hbm, col_hbm, norm_hbm, z_hbm, out_hbm,
               row_v, col_v, norm_v, gbuf, acc, sem):
    cid = lax.axis_index("c")
    sid = lax.axis_index("s")
    pltpu.sync_copy(row_hbm.at[sid], row_v)
    pltpu.sync_copy(col_hbm.at[sid], col_v)
    pltpu.sync_copy(norm_hbm.at[sid], norm_v)
    pltpu.sync_copy(z_hbm, acc.at[pl.ds(sid * ROWS_PER_TILE, ROWS_PER_TILE)])
    plsc.subcore_barrier()

    @pl.loop(0, C2)
    def _chunk(j):
        @pl.when(cid == 0)
        def _g0():
            pltpu.async_copy(xw0_hbm.at[row_v.at[j]], gbuf, sem).wait()

        @pl.when(cid == 1)
        def _g1():
            pltpu.async_copy(xw1_hbm.at[row_v.at[j]], gbuf, sem).wait()

        @pl.loop(0, K // L)
        def _scale(k16):
            nv = norm_v[pl.ds(j * K + k16 * L, L)]
            for kk in range(L):
                s = nv[kk]
                k = k16 * L + kk
                for h in range(HH // L):
                    sl = pl.ds(h * L, L)
                    gbuf[k, sl] = gbuf[k, sl] * s

        pltpu.sync_copy(gbuf, acc.at[col_v.at[j]], add=True)

    plsc.subcore_barrier()
    rs = pl.ds(sid * ROWS_PER_TILE, ROWS_PER_TILE)
    pltpu.sync_copy(acc.at[rs], out_hbm.at[cid, rs])


_scat_kernel = pl.kernel(
    _scat_body,
    out_type=jax.ShapeDtypeStruct((NC, DEG_PAD, HH), jnp.float32),
    mesh=_MESH,
    compiler_params=pltpu.CompilerParams(needs_layout_passes=False,
                                         use_tc_tiling_on_sc=False),
    scratch_types=[
        pltpu.VMEM((C2, K), jnp.int32),
        pltpu.VMEM((C2, K), jnp.int32),
        pltpu.VMEM((EPT2,), jnp.float32),
        pltpu.VMEM((K, HH), jnp.float32),
        pltpu.VMEM_SHARED((DEG_PAD, HH), jnp.float32),
        pltpu.SemaphoreType.DMA,
    ],
)


def _mm_body(x_ref, w_ref, o_ref):
    x = x_ref[...]
    w = w_ref[...]
    o_ref[0, :, :] = jnp.dot(x, w[:, :HH], preferred_element_type=jnp.float32)
    o_ref[1, :, :] = jnp.dot(x, w[:, HH:], preferred_element_type=jnp.float32)


def _mm(x, w):
    return pl.pallas_call(
        _mm_body,
        out_shape=jax.ShapeDtypeStruct((NC, N, HH), jnp.float32),
    )(x, w)


def _p1_body(scat_ref, b_ref, o_ref):
    s = scat_ref[...]
    t = jnp.concatenate([s[0, :N, :], s[1, :N, :]], axis=-1)
    o_ref[...] = t + b_ref[...]


def _p1(scat, b):
    return pl.pallas_call(
        _p1_body,
        out_shape=jax.ShapeDtypeStruct((N, H), jnp.float32),
    )(scat, b.reshape(1, H))


def _ln(t, g, bln):
    mu = jnp.mean(t, axis=-1, keepdims=True)
    var = jnp.mean((t - mu) * (t - mu), axis=-1, keepdims=True)
    return (t - mu) * lax.rsqrt(var + 1e-5) * g + bln


def _pstep_body(scat_ref, b_ref, g_ref, bln_ref, xres_ref, o_ref):
    s = scat_ref[...]
    t = jnp.concatenate([s[0, :N, :], s[1, :N, :]], axis=-1) + b_ref[...]
    y = _ln(t, g_ref[...], bln_ref[...])
    o_ref[...] = jnp.maximum(y + xres_ref[...], 0.0)


def _pstep(scat, b, g, bln, xres):
    return pl.pallas_call(
        _pstep_body,
        out_shape=jax.ShapeDtypeStruct((N, H), jnp.float32),
    )(scat, b.reshape(1, H), g.reshape(1, H), bln.reshape(1, H), xres)


def _pfinal_body(scat_ref, b_ref, g_ref, bln_ref, xres_ref, fcw_ref, fcb_ref,
                 o_ref):
    s = scat_ref[...]
    t = jnp.concatenate([s[0, :N, :], s[1, :N, :]], axis=-1) + b_ref[...]
    y = _ln(t, g_ref[...], bln_ref[...])
    x = jnp.maximum(y + xres_ref[...], 0.0)
    colmean = jnp.sum(x, axis=0, keepdims=True) * (1.0 / N)
    o_ref[...] = jnp.dot(colmean, fcw_ref[...],
                         preferred_element_type=jnp.float32) + fcb_ref[...]


def _pfinal(scat, b, g, bln, xres, fc_W, fc_b):
    return pl.pallas_call(
        _pfinal_body,
        out_shape=jax.ShapeDtypeStruct((1, 2), jnp.float32),
    )(scat, b.reshape(1, H), g.reshape(1, H), bln.reshape(1, H), xres,
      fc_W, fc_b.reshape(1, 2))


def kernel(node, edges, edges_attr, W1, b1, conv_W, conv_b, ln_g, ln_b,
           fc_W, fc_b):
    f32 = jnp.float32
    row = edges[0].astype(jnp.int32)
    col = edges[1].astype(jnp.int32)
    ar = jnp.arange(N, dtype=jnp.int32)
    npad = EP - (E + N)
    zi = jnp.zeros((npad,), jnp.int32)
    row2 = jnp.concatenate([row, ar, zi])
    col2 = jnp.concatenate([col, ar, zi])
    ew2 = jnp.concatenate([edges_attr.astype(f32), jnp.ones((N,), f32),
                           jnp.zeros((npad,), f32)])

    row3 = row2.reshape(NS, C2, K)
    col3 = col2.reshape(NS, C2, K)
    rowf = row2.reshape(NW, EPT)
    colf = col2.reshape(NW, EPT)
    ewf = ew2.reshape(NW, EPT)
    normf2 = None
    zrows = jnp.zeros((ROWS_PER_TILE, HH), f32)

    degp = _deg_kernel(colf, ewf)
    dis = _dis_kernel(degp).reshape(DEG_PAD)
    norm = _norm_kernel(dis, rowf, colf, ewf)

    norm2 = norm.reshape(NS, EPT2)

    def conv(x, W):
        xw = _mm(x, W)
        return _scat_kernel(xw[0], xw[1], row3, col3, norm2, zrows)

    x = _p1(conv(node, W1), b1)
    for i in range(STEPS - 1):
        x = _pstep(conv(x, conv_W[i]), conv_b[i], ln_g[i], ln_b[i], x)
    return _pfinal(conv(x, conv_W[STEPS - 1]), conv_b[STEPS - 1],
                   ln_g[STEPS - 1], ln_b[STEPS - 1], x, fc_W, fc_b)


# speedup vs baseline: 7.5783x; 1.4249x over previous
import functools

import jax
import jax.numpy as jnp
from jax import lax
from jax.experimental import pallas as pl
from jax.experimental.pallas import tpu as pltpu
from jax.experimental.pallas import tpu_sc as plsc

N = 10000
E = 320000
H = 128
STEPS = 3

NC = 2
NS = 16
NW = NC * NS
L = 16

K = 80
C = 129
EPT = C * K
EP = NW * EPT
C2 = 258
EPT2 = C2 * K
HH = H // 2

DEG_PAD = 10240
ROWS_PER_TILE = DEG_PAD // NS

_MESH = plsc.VectorSubcoreMesh(
    core_axis_name="c", subcore_axis_name="s", num_cores=NC, num_subcores=NS)


def _wid():
    return lax.axis_index("c") * NS + lax.axis_index("s")


def _deg_body(col_hbm, ew_hbm, out_hbm, col_v, ew_v, deg_v):
    w = _wid()
    pltpu.sync_copy(col_hbm.at[w], col_v)
    pltpu.sync_copy(ew_hbm.at[w], ew_v)

    @pl.loop(0, DEG_PAD // L)
    def _zero(i):
        deg_v[pl.ds(i * L, L)] = jnp.zeros((L,), jnp.float32)

    @pl.loop(0, EPT // L)
    def _acc(i):
        sl = pl.ds(i * L, L)
        plsc.addupdate_scatter(deg_v, [col_v[sl]], ew_v[sl])

    pltpu.sync_copy(deg_v, out_hbm.at[w])


_deg_kernel = pl.kernel(
    _deg_body,
    out_type=jax.ShapeDtypeStruct((NW, DEG_PAD), jnp.float32),
    mesh=_MESH,
    compiler_params=pltpu.CompilerParams(needs_layout_passes=False),
    scratch_types=[
        pltpu.VMEM((EPT,), jnp.int32),
        pltpu.VMEM((EPT,), jnp.float32),
        pltpu.VMEM((DEG_PAD,), jnp.float32),
    ],
)


def _dis_body(degp_ref, dis_ref):
    deg = jnp.sum(degp_ref[...], axis=0)
    dis_ref[...] = jnp.where(deg > 0.0, lax.rsqrt(deg), 0.0)


def _dis_kernel(degp):
    return pl.pallas_call(
        _dis_body,
        out_shape=jax.ShapeDtypeStruct((DEG_PAD // H, H), jnp.float32),
    )(degp.reshape(NW, DEG_PAD // H, H))


def _norm_body(dis_hbm, row_hbm, col_hbm, ew_hbm, out_hbm,
               dis_v, row_v, col_v, ew_v, norm_v):
    w = _wid()
    pltpu.sync_copy(dis_hbm, dis_v)
    pltpu.sync_copy(row_hbm.at[w], row_v)
    pltpu.sync_copy(col_hbm.at[w], col_v)
    pltpu.sync_copy(ew_hbm.at[w], ew_v)

    @pl.loop(0, EPT // L)
    def _go(i):
        sl = pl.ds(i * L, L)
        gr = plsc.load_gather(dis_v, [row_v[sl]])
        gc = plsc.load_gather(dis_v, [col_v[sl]])
        norm_v[sl] = gr * ew_v[sl] * gc

    pltpu.sync_copy(norm_v, out_hbm.at[w])


_norm_kernel = pl.kernel(
    _norm_body,
    out_type=jax.ShapeDtypeStruct((NW, EPT), jnp.float32),
    mesh=_MESH,
    compiler_params=pltpu.CompilerParams(needs_layout_passes=False),
    scratch_types=[
        pltpu.VMEM((DEG_PAD,), jnp.float32),
        pltpu.VMEM((EPT,), jnp.int32),
        pltpu.VMEM((EPT,), jnp.int32),
        pltpu.VMEM((EPT,), jnp.float32),
        pltpu.VMEM((EPT,), jnp.float32),
    ],
)


def _scat_body(xw0_hbm, xw1_hbm, row_hbm, col_hbm, norm_hbm, z_hbm, out_hbm,
               row_v, col_v, norm_v, gbuf0, gbuf1, acc, semg0, semg1):
    cid = lax.axis_index("c")
    sid = lax.axis_index("s")
    pltpu.sync_copy(row_hbm.at[sid], row_v)
    pltpu.sync_copy(col_hbm.at[sid], col_v)
    pltpu.sync_copy(norm_hbm.at[sid], norm_v)
    pltpu.sync_copy(z_hbm, acc.at[pl.ds(sid * ROWS_PER_TILE, ROWS_PER_TILE)])
    plsc.subcore_barrier()

    def issue_gather(j, buf, sem):
        @pl.when(cid == 0)
        def _g0():
            pltpu.async_copy(xw0_hbm.at[row_v.at[j]], buf, sem)

        @pl.when(cid == 1)
        def _g1():
            pltpu.async_copy(xw1_hbm.at[row_v.at[j]], buf, sem)

    def wait_gather(buf, sem):
        pltpu.make_async_copy(xw0_hbm.at[row_v.at[0]], buf, sem).wait()

    def scale_and_scatter(j, buf):
        @pl.loop(0, K // L)
        def _scale(k16):
            nv = norm_v[pl.ds(j * K + k16 * L, L)]
            for kk in range(L):
                s = nv[kk]
                k = k16 * L + kk
                for h in range(HH // L):
                    sl = pl.ds(h * L, L)
                    buf[k, sl] = buf[k, sl] * s

        pltpu.sync_copy(buf, acc.at[col_v.at[j]], add=True)

    issue_gather(0, gbuf0, semg0)

    @pl.loop(0, C2 // 2)
    def _chunk(i):
        j0 = i * 2
        j1 = j0 + 1
        wait_gather(gbuf0, semg0)
        issue_gather(j1, gbuf1, semg1)
        scale_and_scatter(j0, gbuf0)
        wait_gather(gbuf1, semg1)

        @pl.when(i < C2 // 2 - 1)
        def _pref():
            issue_gather(j0 + 2, gbuf0, semg0)

        scale_and_scatter(j1, gbuf1)

    plsc.subcore_barrier()
    rs = pl.ds(sid * ROWS_PER_TILE, ROWS_PER_TILE)
    pltpu.sync_copy(acc.at[rs], out_hbm.at[cid, rs])


_scat_kernel = pl.kernel(
    _scat_body,
    out_type=jax.ShapeDtypeStruct((NC, DEG_PAD, HH), jnp.float32),
    mesh=_MESH,
    compiler_params=pltpu.CompilerParams(needs_layout_passes=False,
                                         use_tc_tiling_on_sc=False),
    scratch_types=[
        pltpu.VMEM((C2, K), jnp.int32),
        pltpu.VMEM((C2, K), jnp.int32),
        pltpu.VMEM((EPT2,), jnp.float32),
        pltpu.VMEM((K, HH), jnp.float32),
        pltpu.VMEM((K, HH), jnp.float32),
        pltpu.VMEM_SHARED((DEG_PAD, HH), jnp.float32),
        pltpu.SemaphoreType.DMA,
        pltpu.SemaphoreType.DMA,
    ],
)


def _mm_body(x_ref, w_ref, o_ref):
    x = x_ref[...]
    w = w_ref[...]
    o_ref[0, :, :] = jnp.dot(x, w[:, :HH], preferred_element_type=jnp.float32)
    o_ref[1, :, :] = jnp.dot(x, w[:, HH:], preferred_element_type=jnp.float32)


def _mm(x, w):
    return pl.pallas_call(
        _mm_body,
        out_shape=jax.ShapeDtypeStruct((NC, N, HH), jnp.float32),
    )(x, w)


def _p1_body(scat_ref, b_ref, o_ref):
    s = scat_ref[...]
    t = jnp.concatenate([s[0, :N, :], s[1, :N, :]], axis=-1)
    o_ref[...] = t + b_ref[...]


def _p1(scat, b):
    return pl.pallas_call(
        _p1_body,
        out_shape=jax.ShapeDtypeStruct((N, H), jnp.float32),
    )(scat, b.reshape(1, H))


def _ln(t, g, bln):
    mu = jnp.mean(t, axis=-1, keepdims=True)
    var = jnp.mean((t - mu) * (t - mu), axis=-1, keepdims=True)
    return (t - mu) * lax.rsqrt(var + 1e-5) * g + bln


def _pstep_body(scat_ref, b_ref, g_ref, bln_ref, xres_ref, o_ref):
    s = scat_ref[...]
    t = jnp.concatenate([s[0, :N, :], s[1, :N, :]], axis=-1) + b_ref[...]
    y = _ln(t, g_ref[...], bln_ref[...])
    o_ref[...] = jnp.maximum(y + xres_ref[...], 0.0)


def _pstep(scat, b, g, bln, xres):
    return pl.pallas_call(
        _pstep_body,
        out_shape=jax.ShapeDtypeStruct((N, H), jnp.float32),
    )(scat, b.reshape(1, H), g.reshape(1, H), bln.reshape(1, H), xres)


def _pfinal_body(scat_ref, b_ref, g_ref, bln_ref, xres_ref, fcw_ref, fcb_ref,
                 o_ref):
    s = scat_ref[...]
    t = jnp.concatenate([s[0, :N, :], s[1, :N, :]], axis=-1) + b_ref[...]
    y = _ln(t, g_ref[...], bln_ref[...])
    x = jnp.maximum(y + xres_ref[...], 0.0)
    colmean = jnp.sum(x, axis=0, keepdims=True) * (1.0 / N)
    o_ref[...] = jnp.dot(colmean, fcw_ref[...],
                         preferred_element_type=jnp.float32) + fcb_ref[...]


def _pfinal(scat, b, g, bln, xres, fc_W, fc_b):
    return pl.pallas_call(
        _pfinal_body,
        out_shape=jax.ShapeDtypeStruct((1, 2), jnp.float32),
    )(scat, b.reshape(1, H), g.reshape(1, H), bln.reshape(1, H), xres,
      fc_W, fc_b.reshape(1, 2))


def kernel(node, edges, edges_attr, W1, b1, conv_W, conv_b, ln_g, ln_b,
           fc_W, fc_b):
    f32 = jnp.float32
    row = edges[0].astype(jnp.int32)
    col = edges[1].astype(jnp.int32)
    ar = jnp.arange(N, dtype=jnp.int32)
    npad = EP - (E + N)
    zi = jnp.zeros((npad,), jnp.int32)
    row2 = jnp.concatenate([row, ar, zi])
    col2 = jnp.concatenate([col, ar, zi])
    ew2 = jnp.concatenate([edges_attr.astype(f32), jnp.ones((N,), f32),
                           jnp.zeros((npad,), f32)])

    row3 = row2.reshape(NS, C2, K)
    col3 = col2.reshape(NS, C2, K)
    rowf = row2.reshape(NW, EPT)
    colf = col2.reshape(NW, EPT)
    ewf = ew2.reshape(NW, EPT)
    normf2 = None
    zrows = jnp.zeros((ROWS_PER_TILE, HH), f32)

    degp = _deg_kernel(colf, ewf)
    dis = _dis_kernel(degp).reshape(DEG_PAD)
    norm = _norm_kernel(dis, rowf, colf, ewf)

    norm2 = norm.reshape(NS, EPT2)

    def conv(x, W):
        xw = _mm(x, W)
        return _scat_kernel(xw[0], xw[1], row3, col3, norm2, zrows)

    x = _p1(conv(node, W1), b1)
    for i in range(STEPS - 1):
        x = _pstep(conv(x, conv_W[i]), conv_b[i], ln_g[i], ln_b[i], x)
    return _pfinal(conv(x, conv_W[STEPS - 1]), conv_b[STEPS - 1],
                   ln_g[STEPS - 1], ln_b[STEPS - 1], x, fc_W, fc_b)
